# baseline (speedup 1.0000x reference)
import functools

import jax
import jax.numpy as jnp
from jax import lax
from jax.experimental import pallas as pl
from jax.experimental.pallas import tpu as pltpu
from jax.experimental.pallas import tpu_sc as plsc

_D = 64
_CH = 128
_C = 2
_R = _C * _CH
_NW = 32


def _gather_rows(xeo, table):
    lines = xeo.shape[1] * _CH
    lines_per_w = lines // _NW
    rounds = lines_per_w // _R
    assert lines_per_w % _R == 0 and rounds % 2 == 0 and rounds >= 4
    idxrows_per_w = lines_per_w // _CH

    mesh = plsc.VectorSubcoreMesh(core_axis_name="c", subcore_axis_name="s")

    @functools.partial(
        pl.kernel,
        mesh=mesh,
        out_type=jax.ShapeDtypeStruct((lines, 2 * _D), jnp.float32),
        scratch_types=[
            pltpu.VMEM((2, idxrows_per_w, _CH), jnp.int32),
            pltpu.VMEM((2, _R, _D), jnp.float32),
            pltpu.VMEM((2, _R, _D), jnp.float32),
            pltpu.SemaphoreType.DMA,
            pltpu.SemaphoreType.DMA,
            pltpu.SemaphoreType.DMA,
            pltpu.SemaphoreType.DMA,
        ],
        compiler_params=pltpu.CompilerParams(use_tc_tiling_on_sc=False),
    )
    def body(idx_hbm, table_hbm, out_hbm, idx_v, rows0, rows1,
             gsem0, gsem1, ssem0, ssem1):
        wid = lax.axis_index("s") * 2 + lax.axis_index("c")
        out_base = wid * lines_per_w
        pltpu.sync_copy(
            idx_hbm.at[:, pl.ds(wid * idxrows_per_w, idxrows_per_w)], idx_v)

        def fire_gathers(r, grp, gsem):
            for c in range(_C):
                k = r * _C + c
                for h in range(2):
                    pltpu.async_copy(
                        table_hbm.at[idx_v.at[h, k]],
                        grp.at[h, pl.ds(c * _CH, _CH)], gsem)

        def wait_gathers(r, grp, gsem):
            for c in range(_C):
                k = r * _C + c
                for h in range(2):
                    pltpu.make_async_copy(
                        table_hbm.at[idx_v.at[h, k]],
                        grp.at[h, pl.ds(c * _CH, _CH)], gsem).wait()

        def fire_scatter(r, grp, ssem):
            for h in range(2):
                pltpu.async_copy(
                    grp.at[h],
                    out_hbm.at[pl.ds(out_base + r * _R, _R),
                               pl.ds(h * _D, _D)], ssem)

        def wait_scatter(r, grp, ssem):
            for h in range(2):
                pltpu.make_async_copy(
                    grp.at[h],
                    out_hbm.at[pl.ds(out_base + r * _R, _R),
                               pl.ds(h * _D, _D)], ssem).wait()

        fire_gathers(0, rows0, gsem0)
        fire_gathers(1, rows1, gsem1)
        wait_gathers(0, rows0, gsem0)
        fire_scatter(0, rows0, ssem0)

        @pl.loop(0, (rounds - 2) // 2)
        def _steady(i):
            r = 1 + 2 * i
            wait_scatter(r - 1, rows0, ssem0)
            fire_gathers(r + 1, rows0, gsem0)
            wait_gathers(r, rows1, gsem1)
            fire_scatter(r, rows1, ssem1)
            wait_scatter(r, rows1, ssem1)
            fire_gathers(r + 2, rows1, gsem1)
            wait_gathers(r + 1, rows0, gsem0)
            fire_scatter(r + 1, rows0, ssem0)

        r_last = rounds - 1
        wait_scatter(r_last - 1, rows0, ssem0)
        wait_gathers(r_last, rows1, gsem1)
        fire_scatter(r_last, rows1, ssem1)
        wait_scatter(r_last, rows1, ssem1)

    return body(xeo, table)


def kernel(x, pe):
    b, l = x.shape
    xp = x.reshape(-1, 2)
    xeo = xp.T.reshape(2, -1, _CH)
    out = _gather_rows(xeo, pe)
    return out.reshape(b, l, _D)
